# Initial kernel scaffold; baseline (speedup 1.0000x reference)
import jax
import jax.numpy as jnp
from jax.experimental import pallas as pl


def kernel(x, edge_index, W1, b1, W_mu, b_mu, W_lv, b_lv):
    raise NotImplementedError("write your pallas kernel here")


# speedup vs baseline: 24.2564x; 24.2564x over previous
import functools

import jax
import jax.numpy as jnp
from jax import lax
from jax.experimental import pallas as pl
from jax.experimental.pallas import tpu as pltpu
from jax.experimental.pallas import tpu_sc as plsc

N_NODES = 10000
N_EDGES = 320000
IN_CH = 128
HID = 32
OUT_CH = 16

NPAD = 10240
N_WORKERS = 32
CHUNK = 128
N_CHUNKS = 80
EPAD = N_WORKERS * N_CHUNKS * CHUNK
ROWS_PER_TILE = NPAD // 16

_MESH = plsc.VectorSubcoreMesh(core_axis_name="c", subcore_axis_name="s")
_SC_PARAMS = pltpu.CompilerParams(use_tc_tiling_on_sc=False)


@functools.partial(
    pl.kernel,
    out_type=jax.ShapeDtypeStruct((2, NPAD), jnp.float32),
    mesh=_MESH,
    compiler_params=_SC_PARAMS,
    scratch_types=[
        pltpu.VMEM((N_CHUNKS, CHUNK), jnp.int32),
        pltpu.VMEM((CHUNK,), jnp.float32),
        pltpu.VMEM_SHARED((NPAD,), jnp.float32),
    ],
)
def _sc_degree(dst_hbm, zeros_hbm, out_hbm, dst_v, ones_v, acc_sh):
    cid = lax.axis_index("c")
    sid = lax.axis_index("s")
    wid = sid * 2 + cid
    base = sid * ROWS_PER_TILE
    pltpu.sync_copy(zeros_hbm.at[pl.ds(base, ROWS_PER_TILE)],
                    acc_sh.at[pl.ds(base, ROWS_PER_TILE)])
    pltpu.sync_copy(dst_hbm.at[wid], dst_v)
    for i in range(CHUNK // 16):
        ones_v[pl.ds(i * 16, 16)] = jnp.ones((16,), jnp.float32)
    plsc.subcore_barrier()

    @pl.loop(0, N_CHUNKS)
    def _(j):
        pltpu.sync_copy(ones_v, acc_sh.at[dst_v.at[j]], add=True)

    plsc.subcore_barrier()
    pltpu.sync_copy(acc_sh.at[pl.ds(base, ROWS_PER_TILE)],
                    out_hbm.at[cid, pl.ds(base, ROWS_PER_TILE)])


@functools.partial(
    pl.kernel,
    out_type=jax.ShapeDtypeStruct((2, NPAD, HID), jnp.float32),
    mesh=_MESH,
    compiler_params=_SC_PARAMS,
    scratch_types=[
        pltpu.VMEM((N_CHUNKS, CHUNK), jnp.int32),
        pltpu.VMEM((N_CHUNKS, CHUNK), jnp.int32),
        pltpu.VMEM((CHUNK, HID), jnp.float32),
        pltpu.VMEM_SHARED((NPAD, HID), jnp.float32),
        pltpu.SemaphoreType.DMA,
    ],
)
def _sc_aggregate(g_hbm, src_hbm, dst_hbm, zeros_hbm, out_hbm,
                  src_v, dst_v, rows_v, acc_sh, sem):
    cid = lax.axis_index("c")
    sid = lax.axis_index("s")
    wid = sid * 2 + cid
    base = sid * ROWS_PER_TILE
    pltpu.sync_copy(zeros_hbm.at[pl.ds(base, ROWS_PER_TILE)],
                    acc_sh.at[pl.ds(base, ROWS_PER_TILE)])
    pltpu.sync_copy(src_hbm.at[wid], src_v)
    pltpu.sync_copy(dst_hbm.at[wid], dst_v)
    plsc.subcore_barrier()

    @pl.loop(0, N_CHUNKS)
    def _(j):
        pltpu.async_copy(g_hbm.at[src_v.at[j]], rows_v, sem).wait()
        pltpu.sync_copy(rows_v, acc_sh.at[dst_v.at[j]], add=True)

    plsc.subcore_barrier()
    pltpu.sync_copy(acc_sh.at[pl.ds(base, ROWS_PER_TILE)],
                    out_hbm.at[cid, pl.ds(base, ROWS_PER_TILE)])


_BLK = 1024
_GRID = NPAD // _BLK


def _dinv_of(degp):
    return lax.rsqrt(degp[0] + degp[1] + 1.0)


def _tc_prep_body(x_ref, w1_ref, degp_ref, g1_ref):
    t = jnp.dot(x_ref[...], w1_ref[...], preferred_element_type=jnp.float32)
    dinv = _dinv_of(degp_ref[...])
    g1_ref[...] = t * dinv[:, None]


def _tc_layer1_body(parts_ref, g1_ref, degp_ref, b1_ref, g2_ref):
    dinv = _dinv_of(degp_ref[...])
    parts = parts_ref[...]
    aggr = dinv[:, None] * (parts[0] + parts[1] + g1_ref[...])
    h = jnp.maximum(aggr + b1_ref[...], 0.0)
    g2_ref[...] = h * dinv[:, None]


def _tc_final_body(parts_ref, g2_ref, degp_ref, wmu_ref, bmu_ref,
                   wlv_ref, blv_ref, mu_ref, lv_ref):
    dinv = _dinv_of(degp_ref[...])
    parts = parts_ref[...]
    z = dinv[:, None] * (parts[0] + parts[1] + g2_ref[...])
    mu_ref[...] = jnp.dot(z, wmu_ref[...],
                          preferred_element_type=jnp.float32) + bmu_ref[...]
    lv_ref[...] = jnp.dot(z, wlv_ref[...],
                          preferred_element_type=jnp.float32) + blv_ref[...]


def _row_spec(width):
    return pl.BlockSpec((_BLK, width), lambda i: (i, 0))


def _full_spec(shape):
    return pl.BlockSpec(shape, lambda i: (0,) * len(shape))


_degp_spec = pl.BlockSpec((2, _BLK), lambda i: (0, i))
_parts_spec = pl.BlockSpec((2, _BLK, HID), lambda i: (0, i, 0))


def _tc_prep(x_p, W1, degp):
    return pl.pallas_call(
        _tc_prep_body,
        grid=(_GRID,),
        in_specs=[_row_spec(IN_CH), _full_spec((IN_CH, HID)), _degp_spec],
        out_specs=_row_spec(HID),
        out_shape=jax.ShapeDtypeStruct((NPAD, HID), jnp.float32),
    )(x_p, W1, degp)


def _tc_layer1(parts, g1, degp, b1):
    return pl.pallas_call(
        _tc_layer1_body,
        grid=(_GRID,),
        in_specs=[_parts_spec, _row_spec(HID), _degp_spec,
                  _full_spec((1, HID))],
        out_specs=_row_spec(HID),
        out_shape=jax.ShapeDtypeStruct((NPAD, HID), jnp.float32),
    )(parts, g1, degp, b1)


def _tc_final(parts, g2, degp, W_mu, b_mu, W_lv, b_lv):
    return pl.pallas_call(
        _tc_final_body,
        grid=(_GRID,),
        in_specs=[_parts_spec, _row_spec(HID), _degp_spec,
                  _full_spec((HID, OUT_CH)), _full_spec((1, OUT_CH)),
                  _full_spec((HID, OUT_CH)), _full_spec((1, OUT_CH))],
        out_specs=[_row_spec(OUT_CH), _row_spec(OUT_CH)],
        out_shape=[jax.ShapeDtypeStruct((NPAD, OUT_CH), jnp.float32),
                   jax.ShapeDtypeStruct((NPAD, OUT_CH), jnp.float32)],
    )(parts, g2, degp, W_mu, b_mu, W_lv, b_lv)


def kernel(x, edge_index, W1, b1, W_mu, b_mu, W_lv, b_lv):
    src = edge_index[0].astype(jnp.int32)
    dst = edge_index[1].astype(jnp.int32)
    npad_e = EPAD - N_EDGES
    src3 = jnp.concatenate(
        [src, jnp.zeros((npad_e,), jnp.int32)]).reshape(N_WORKERS, N_CHUNKS, CHUNK)
    dst3 = jnp.concatenate(
        [dst, jnp.full((npad_e,), N_NODES, jnp.int32)]).reshape(N_WORKERS, N_CHUNKS, CHUNK)
    x_p = jnp.pad(x, ((0, NPAD - N_NODES), (0, 0)))
    z1 = jnp.zeros((NPAD,), jnp.float32)
    z2 = jnp.zeros((NPAD, HID), jnp.float32)

    degp = _sc_degree(dst3, z1)
    g1 = _tc_prep(x_p, W1, degp)
    parts1 = _sc_aggregate(g1, src3, dst3, z2)
    g2 = _tc_layer1(parts1, g1, degp, b1.reshape(1, HID))
    parts2 = _sc_aggregate(g2, src3, dst3, z2)
    mu, lv = _tc_final(parts2, g2, degp, W_mu, b_mu.reshape(1, OUT_CH),
                       W_lv, b_lv.reshape(1, OUT_CH))
    return (mu[:N_NODES], lv[:N_NODES])


# speedup vs baseline: 29.9293x; 1.2339x over previous
import functools

import jax
import jax.numpy as jnp
from jax import lax
from jax.experimental import pallas as pl
from jax.experimental.pallas import tpu as pltpu
from jax.experimental.pallas import tpu_sc as plsc

N_NODES = 10000
N_EDGES = 320000
IN_CH = 128
HID = 32
OUT_CH = 16

NPAD = 10240
N_WORKERS = 32
CHUNK = 128
N_CHUNKS = 80
EPAD = N_WORKERS * N_CHUNKS * CHUNK
ROWS_PER_TILE = NPAD // 16

_MESH = plsc.VectorSubcoreMesh(core_axis_name="c", subcore_axis_name="s")
_SC_PARAMS = pltpu.CompilerParams(use_tc_tiling_on_sc=False)


@functools.partial(
    pl.kernel,
    out_type=jax.ShapeDtypeStruct((2, NPAD), jnp.float32),
    mesh=_MESH,
    compiler_params=_SC_PARAMS,
    scratch_types=[
        pltpu.VMEM((N_CHUNKS, CHUNK), jnp.int32),
        pltpu.VMEM((CHUNK,), jnp.float32),
        pltpu.VMEM_SHARED((NPAD,), jnp.float32),
    ],
)
def _sc_degree(dst_hbm, zeros_hbm, out_hbm, dst_v, ones_v, acc_sh):
    cid = lax.axis_index("c")
    sid = lax.axis_index("s")
    wid = sid * 2 + cid
    base = sid * ROWS_PER_TILE
    pltpu.sync_copy(zeros_hbm.at[pl.ds(base, ROWS_PER_TILE)],
                    acc_sh.at[pl.ds(base, ROWS_PER_TILE)])
    pltpu.sync_copy(dst_hbm.at[wid], dst_v)
    for i in range(CHUNK // 16):
        ones_v[pl.ds(i * 16, 16)] = jnp.ones((16,), jnp.float32)
    plsc.subcore_barrier()

    @pl.loop(0, N_CHUNKS)
    def _(j):
        pltpu.sync_copy(ones_v, acc_sh.at[dst_v.at[j]], add=True)

    plsc.subcore_barrier()
    pltpu.sync_copy(acc_sh.at[pl.ds(base, ROWS_PER_TILE)],
                    out_hbm.at[cid, pl.ds(base, ROWS_PER_TILE)])


@functools.partial(
    pl.kernel,
    out_type=jax.ShapeDtypeStruct((2, NPAD, HID), jnp.float32),
    mesh=_MESH,
    compiler_params=_SC_PARAMS,
    scratch_types=[
        pltpu.VMEM((N_CHUNKS, CHUNK), jnp.int32),
        pltpu.VMEM((N_CHUNKS, CHUNK), jnp.int32),
        pltpu.VMEM((2, CHUNK, HID), jnp.float32),
        pltpu.VMEM_SHARED((NPAD, HID), jnp.float32),
        pltpu.SemaphoreType.DMA,
        pltpu.SemaphoreType.DMA,
    ],
)
def _sc_aggregate(g_hbm, src_hbm, dst_hbm, zeros_hbm, out_hbm,
                  src_v, dst_v, rows_v, acc_sh, sem0, sem1):
    cid = lax.axis_index("c")
    sid = lax.axis_index("s")
    wid = sid * 2 + cid
    base = sid * ROWS_PER_TILE
    pltpu.sync_copy(zeros_hbm.at[pl.ds(base, ROWS_PER_TILE)],
                    acc_sh.at[pl.ds(base, ROWS_PER_TILE)])
    pltpu.sync_copy(src_hbm.at[wid], src_v)
    pltpu.sync_copy(dst_hbm.at[wid], dst_v)
    plsc.subcore_barrier()

    pltpu.async_copy(g_hbm.at[src_v.at[0]], rows_v.at[0], sem0)

    @pl.loop(0, N_CHUNKS, step=2)
    def _(j):
        pltpu.async_copy(g_hbm.at[src_v.at[j + 1]], rows_v.at[1], sem1)
        pltpu.make_async_copy(g_hbm.at[src_v.at[j]], rows_v.at[0], sem0).wait()
        pltpu.sync_copy(rows_v.at[0], acc_sh.at[dst_v.at[j]], add=True)

        @pl.when(j + 2 < N_CHUNKS)
        def _():
            pltpu.async_copy(g_hbm.at[src_v.at[j + 2]], rows_v.at[0], sem0)

        pltpu.make_async_copy(g_hbm.at[src_v.at[j + 1]], rows_v.at[1],
                              sem1).wait()
        pltpu.sync_copy(rows_v.at[1], acc_sh.at[dst_v.at[j + 1]], add=True)

    plsc.subcore_barrier()
    pltpu.sync_copy(acc_sh.at[pl.ds(base, ROWS_PER_TILE)],
                    out_hbm.at[cid, pl.ds(base, ROWS_PER_TILE)])


_BLK = 1024
_GRID = NPAD // _BLK


def _dinv_of(degp):
    return lax.rsqrt(degp[0] + degp[1] + 1.0)


def _tc_prep_body(x_ref, w1_ref, degp_ref, g1_ref):
    t = jnp.dot(x_ref[...], w1_ref[...], preferred_element_type=jnp.float32)
    dinv = _dinv_of(degp_ref[...])
    g1_ref[...] = t * dinv[:, None]


def _tc_layer1_body(parts_ref, g1_ref, degp_ref, b1_ref, g2_ref):
    dinv = _dinv_of(degp_ref[...])
    parts = parts_ref[...]
    aggr = dinv[:, None] * (parts[0] + parts[1] + g1_ref[...])
    h = jnp.maximum(aggr + b1_ref[...], 0.0)
    g2_ref[...] = h * dinv[:, None]


def _tc_final_body(parts_ref, g2_ref, degp_ref, wmu_ref, bmu_ref,
                   wlv_ref, blv_ref, mu_ref, lv_ref):
    dinv = _dinv_of(degp_ref[...])
    parts = parts_ref[...]
    z = dinv[:, None] * (parts[0] + parts[1] + g2_ref[...])
    mu_ref[...] = jnp.dot(z, wmu_ref[...],
                          preferred_element_type=jnp.float32) + bmu_ref[...]
    lv_ref[...] = jnp.dot(z, wlv_ref[...],
                          preferred_element_type=jnp.float32) + blv_ref[...]


def _row_spec(width):
    return pl.BlockSpec((_BLK, width), lambda i: (i, 0))


def _full_spec(shape):
    return pl.BlockSpec(shape, lambda i: (0,) * len(shape))


_degp_spec = pl.BlockSpec((2, _BLK), lambda i: (0, i))
_parts_spec = pl.BlockSpec((2, _BLK, HID), lambda i: (0, i, 0))


def _tc_prep(x_p, W1, degp):
    return pl.pallas_call(
        _tc_prep_body,
        grid=(_GRID,),
        in_specs=[_row_spec(IN_CH), _full_spec((IN_CH, HID)), _degp_spec],
        out_specs=_row_spec(HID),
        out_shape=jax.ShapeDtypeStruct((NPAD, HID), jnp.float32),
    )(x_p, W1, degp)


def _tc_layer1(parts, g1, degp, b1):
    return pl.pallas_call(
        _tc_layer1_body,
        grid=(_GRID,),
        in_specs=[_parts_spec, _row_spec(HID), _degp_spec,
                  _full_spec((1, HID))],
        out_specs=_row_spec(HID),
        out_shape=jax.ShapeDtypeStruct((NPAD, HID), jnp.float32),
    )(parts, g1, degp, b1)


def _tc_final(parts, g2, degp, W_mu, b_mu, W_lv, b_lv):
    return pl.pallas_call(
        _tc_final_body,
        grid=(_GRID,),
        in_specs=[_parts_spec, _row_spec(HID), _degp_spec,
                  _full_spec((HID, OUT_CH)), _full_spec((1, OUT_CH)),
                  _full_spec((HID, OUT_CH)), _full_spec((1, OUT_CH))],
        out_specs=[_row_spec(OUT_CH), _row_spec(OUT_CH)],
        out_shape=[jax.ShapeDtypeStruct((NPAD, OUT_CH), jnp.float32),
                   jax.ShapeDtypeStruct((NPAD, OUT_CH), jnp.float32)],
    )(parts, g2, degp, W_mu, b_mu, W_lv, b_lv)


def kernel(x, edge_index, W1, b1, W_mu, b_mu, W_lv, b_lv):
    src = edge_index[0].astype(jnp.int32)
    dst = edge_index[1].astype(jnp.int32)
    npad_e = EPAD - N_EDGES
    src3 = jnp.concatenate(
        [src, jnp.zeros((npad_e,), jnp.int32)]).reshape(N_WORKERS, N_CHUNKS, CHUNK)
    dst3 = jnp.concatenate(
        [dst, jnp.full((npad_e,), N_NODES, jnp.int32)]).reshape(N_WORKERS, N_CHUNKS, CHUNK)
    x_p = jnp.pad(x, ((0, NPAD - N_NODES), (0, 0)))
    z1 = jnp.zeros((NPAD,), jnp.float32)
    z2 = jnp.zeros((NPAD, HID), jnp.float32)

    degp = _sc_degree(dst3, z1)
    g1 = _tc_prep(x_p, W1, degp)
    parts1 = _sc_aggregate(g1, src3, dst3, z2)
    g2 = _tc_layer1(parts1, g1, degp, b1.reshape(1, HID))
    parts2 = _sc_aggregate(g2, src3, dst3, z2)
    mu, lv = _tc_final(parts2, g2, degp, W_mu, b_mu.reshape(1, OUT_CH),
                       W_lv, b_lv.reshape(1, OUT_CH))
    return (mu[:N_NODES], lv[:N_NODES])
